# Initial kernel scaffold; baseline (speedup 1.0000x reference)
import jax
import jax.numpy as jnp
from jax.experimental import pallas as pl


def kernel(emb_table, indices):
    raise NotImplementedError("write your pallas kernel here")


# speedup vs baseline: 1.0610x; 1.0610x over previous
import functools

import jax
import jax.numpy as jnp
from jax import lax
from jax.experimental import pallas as pl
from jax.experimental.pallas import tpu as pltpu
from jax.experimental.pallas import tpu_sc as plsc

EMB = 32
CHUNK = 128
NUM_WORKERS = 32


@functools.cache
def _build(B):
    assert B % (NUM_WORKERS * CHUNK) == 0
    b_per_w = B // NUM_WORKERS
    nchunks = b_per_w // CHUNK
    mesh = plsc.VectorSubcoreMesh(core_axis_name="c", subcore_axis_name="s")

    @functools.partial(
        pl.kernel,
        mesh=mesh,
        out_type=jax.ShapeDtypeStruct((B, EMB), jnp.float32),
        scratch_types=[
            pltpu.VMEM((b_per_w,), jnp.int32),
            pltpu.VMEM((CHUNK, EMB), jnp.float32),
            pltpu.SemaphoreType.DMA,
        ],
        compiler_params=pltpu.CompilerParams(use_tc_tiling_on_sc=False),
    )
    def gather_kernel(table_hbm, idx_hbm, out_hbm, idx_v, rows_v, sem):
        wid = lax.axis_index("s") * 2 + lax.axis_index("c")
        base = wid * b_per_w
        pltpu.sync_copy(idx_hbm.at[pl.ds(base, b_per_w)], idx_v)

        def body(g, carry):
            off = g * CHUNK
            pltpu.async_copy(
                table_hbm.at[idx_v.at[pl.ds(off, CHUNK)]], rows_v, sem
            ).wait()
            pltpu.sync_copy(rows_v, out_hbm.at[pl.ds(base + off, CHUNK)])
            return carry

        lax.fori_loop(0, nchunks, body, 0)

    return gather_kernel


def kernel(emb_table, indices):
    shape = indices.shape
    flat = indices.reshape(-1).astype(jnp.int32)
    out = _build(flat.shape[0])(emb_table, flat)
    return out.reshape(*shape, EMB)


# speedup vs baseline: 1.1122x; 1.0482x over previous
import functools

import jax
import jax.numpy as jnp
from jax import lax
from jax.experimental import pallas as pl
from jax.experimental.pallas import tpu as pltpu
from jax.experimental.pallas import tpu_sc as plsc

EMB = 32
CHUNK = 128
NBUF = 8
NUM_WORKERS = 32


@functools.cache
def _build(B):
    assert B % (NUM_WORKERS * CHUNK * NBUF) == 0
    b_per_w = B // NUM_WORKERS
    nchunks = b_per_w // CHUNK
    mesh = plsc.VectorSubcoreMesh(core_axis_name="c", subcore_axis_name="s")

    @functools.partial(
        pl.kernel,
        mesh=mesh,
        out_type=jax.ShapeDtypeStruct((B, EMB), jnp.float32),
        scratch_types=[
            pltpu.VMEM((b_per_w,), jnp.int32),
            [pltpu.VMEM((CHUNK, EMB), jnp.float32) for _ in range(NBUF)],
            [pltpu.SemaphoreType.DMA for _ in range(NBUF)],
            [pltpu.SemaphoreType.DMA for _ in range(NBUF)],
        ],
        compiler_params=pltpu.CompilerParams(use_tc_tiling_on_sc=False),
    )
    def gather_kernel(table_hbm, idx_hbm, out_hbm, idx_v, bufs, gsems, osems):
        wid = lax.axis_index("s") * 2 + lax.axis_index("c")
        base = wid * b_per_w
        pltpu.sync_copy(idx_hbm.at[pl.ds(base, b_per_w)], idx_v)

        def g_start(g, b):
            pltpu.async_copy(
                table_hbm.at[idx_v.at[pl.ds(g * CHUNK, CHUNK)]], bufs[b], gsems[b]
            )

        def g_wait(b):
            pltpu.make_async_copy(
                table_hbm.at[idx_v.at[pl.ds(0, CHUNK)]], bufs[b], gsems[b]
            ).wait()

        def o_start(g, b):
            pltpu.async_copy(
                bufs[b], out_hbm.at[pl.ds(base + g * CHUNK, CHUNK)], osems[b]
            )

        def o_wait(b):
            pltpu.make_async_copy(
                bufs[b], out_hbm.at[pl.ds(base, CHUNK)], osems[b]
            ).wait()

        for b in range(NBUF):
            g_start(b, b)

        def body(i, carry):
            g0 = i * NBUF
            for b in range(NBUF):
                g_wait(b)
                o_start(g0 + b, b)
            for b in range(NBUF):
                o_wait(b)

                @pl.when(g0 + b + NBUF < nchunks)
                def _():
                    g_start(g0 + b + NBUF, b)

            return carry

        lax.fori_loop(0, nchunks // NBUF, body, 0)

    return gather_kernel


def kernel(emb_table, indices):
    shape = indices.shape
    flat = indices.reshape(-1).astype(jnp.int32)
    out = _build(flat.shape[0])(emb_table, flat)
    return out.reshape(*shape, EMB)


# speedup vs baseline: 2.6860x; 2.4152x over previous
import functools

import jax
import jax.numpy as jnp
from jax import lax
from jax.experimental import pallas as pl
from jax.experimental.pallas import tpu as pltpu
from jax.experimental.pallas import tpu_sc as plsc

EMB = 32
CHUNK = 128
NUM_WORKERS = 32
LANES = 16


@functools.cache
def _build(S, B):
    nb = B // CHUNK
    nq = S * nb
    assert nq % NUM_WORKERS == 0
    q_per_w = nq // NUM_WORKERS
    mesh = plsc.VectorSubcoreMesh(core_axis_name="c", subcore_axis_name="s")

    @functools.partial(
        pl.kernel,
        mesh=mesh,
        out_type=jax.ShapeDtypeStruct((S, EMB, B), jnp.float32),
        scratch_types=[
            pltpu.VMEM((CHUNK,), jnp.int32),
            pltpu.VMEM((CHUNK, EMB), jnp.float32),
            pltpu.VMEM((EMB, CHUNK), jnp.float32),
            pltpu.SemaphoreType.DMA,
        ],
        compiler_params=pltpu.CompilerParams(
            use_tc_tiling_on_sc=False, needs_layout_passes=False
        ),
    )
    def gather_kernel(table_hbm, idx_hbm, out_hbm, idx_v, gbuf, tbuf, sem):
        wid = lax.axis_index("s") * 2 + lax.axis_index("c")
        q0 = wid * q_per_w

        def body(i, carry):
            q = q0 + i
            s = q // nb
            b0 = (q % nb) * CHUNK
            pltpu.sync_copy(idx_hbm.at[s, pl.ds(b0, CHUNK)], idx_v)
            pltpu.async_copy(table_hbm.at[idx_v], gbuf, sem).wait()
            for c in range(EMB):
                col = jnp.full((LANES,), c, dtype=jnp.int32)
                for grp in range(CHUNK // LANES):
                    rows = lax.iota(jnp.int32, LANES) + grp * LANES
                    vals = plsc.load_gather(gbuf, [rows, col])
                    tbuf[c, pl.ds(grp * LANES, LANES)] = vals
            pltpu.sync_copy(tbuf, out_hbm.at[s, :, pl.ds(b0, CHUNK)])
            return carry

        lax.fori_loop(0, q_per_w, body, 0)

    return gather_kernel


def kernel(emb_table, indices):
    Bn, Sn = indices.shape
    idx_t = indices.T.astype(jnp.int32)
    out_t = _build(Sn, Bn)(emb_table, idx_t)
    return out_t.transpose(2, 0, 1)


# speedup vs baseline: 3.1382x; 1.1683x over previous
import functools

import jax
import jax.numpy as jnp
from jax import lax
from jax.experimental import pallas as pl
from jax.experimental.pallas import tpu as pltpu
from jax.experimental.pallas import tpu_sc as plsc

EMB = 32
CHUNK = 128
NBUF = 4
NUM_WORKERS = 32
LANES = 16
GRPS = CHUNK // LANES


@functools.cache
def _build(S, B):
    nq = (S * B) // CHUNK
    assert nq % (NUM_WORKERS * NBUF) == 0
    q_per_w = nq // NUM_WORKERS
    n_per_w = q_per_w * CHUNK
    mesh = plsc.VectorSubcoreMesh(core_axis_name="c", subcore_axis_name="s")

    @functools.partial(
        pl.kernel,
        mesh=mesh,
        out_type=jax.ShapeDtypeStruct((S, EMB, B), jnp.float32),
        scratch_types=[
            pltpu.VMEM((n_per_w,), jnp.int32),
            [pltpu.VMEM((CHUNK, EMB), jnp.float32) for _ in range(NBUF)],
            [pltpu.VMEM((EMB, CHUNK), jnp.float32) for _ in range(NBUF)],
            [pltpu.SemaphoreType.DMA for _ in range(NBUF)],
            [pltpu.SemaphoreType.DMA for _ in range(NBUF)],
        ],
        compiler_params=pltpu.CompilerParams(
            use_tc_tiling_on_sc=False, needs_layout_passes=False
        ),
    )
    def gather_kernel(table_hbm, idx_hbm, out_hbm, idx_v, gbufs, tbufs, gsems, osems):
        wid = lax.axis_index("s") * 2 + lax.axis_index("c")
        q0 = wid * q_per_w
        pltpu.sync_copy(idx_hbm.at[pl.ds(q0 * CHUNK, n_per_w)], idx_v)

        nb = B // CHUNK

        def g_start(g, b):
            pltpu.async_copy(
                table_hbm.at[idx_v.at[pl.ds(g * CHUNK, CHUNK)]], gbufs[b], gsems[b]
            )

        def g_wait(b):
            pltpu.make_async_copy(
                table_hbm.at[idx_v.at[pl.ds(0, CHUNK)]], gbufs[b], gsems[b]
            ).wait()

        def o_start(q, b):
            s = q // nb
            b0 = (q % nb) * CHUNK
            pltpu.async_copy(
                tbufs[b], out_hbm.at[s, :, pl.ds(b0, CHUNK)], osems[b]
            )

        def o_wait(b):
            pltpu.make_async_copy(
                tbufs[b], out_hbm.at[0, :, pl.ds(0, CHUNK)], osems[b]
            ).wait()

        rows = [lax.iota(jnp.int32, LANES) + grp * LANES for grp in range(GRPS)]

        def transpose(b):
            gbuf, tbuf = gbufs[b], tbufs[b]
            for c in range(EMB):
                csplat = jnp.full((LANES,), c, jnp.int32)
                for grp in range(GRPS):
                    vals = plsc.load_gather(gbuf, [rows[grp], csplat])
                    tbuf[c, pl.ds(grp * LANES, LANES)] = vals

        for b in range(NBUF):
            g_start(b, b)

        def body(i, carry):
            g0 = i * NBUF
            for b in range(NBUF):
                g = g0 + b
                g_wait(b)

                @pl.when(i > 0)
                def _():
                    o_wait(b)

                transpose(b)

                @pl.when(g + NBUF < q_per_w)
                def _():
                    g_start(g + NBUF, b)

                o_start(q0 + g, b)
            return carry

        lax.fori_loop(0, q_per_w // NBUF, body, 0)
        for b in range(NBUF):
            o_wait(b)

    return gather_kernel


def kernel(emb_table, indices):
    Bn, Sn = indices.shape
    idx_flat = indices.T.reshape(-1).astype(jnp.int32)
    out_t = _build(Sn, Bn)(emb_table, idx_flat)
    return out_t.transpose(2, 0, 1)


# speedup vs baseline: 3.4825x; 1.1097x over previous
import functools

import jax
import jax.numpy as jnp
from jax import lax
from jax.experimental import pallas as pl
from jax.experimental.pallas import tpu as pltpu
from jax.experimental.pallas import tpu_sc as plsc

EMB = 32
CHUNK = 128
GRP_CHUNKS = 4
GROUP = CHUNK * GRP_CHUNKS
NBUF = 2
NUM_WORKERS = 32
LANES = 16


@functools.cache
def _build(S, B):
    ng = (S * B) // GROUP
    assert ng % (NUM_WORKERS * NBUF) == 0 and B % GROUP == 0
    g_per_w = ng // NUM_WORKERS
    n_per_w = g_per_w * GROUP
    mesh = plsc.VectorSubcoreMesh(core_axis_name="c", subcore_axis_name="s")

    @functools.partial(
        pl.kernel,
        mesh=mesh,
        out_type=jax.ShapeDtypeStruct((S, EMB, B), jnp.float32),
        scratch_types=[
            pltpu.VMEM((n_per_w,), jnp.int32),
            [pltpu.VMEM((GROUP, EMB), jnp.float32) for _ in range(NBUF)],
            [pltpu.VMEM((EMB, GROUP), jnp.float32) for _ in range(NBUF)],
            [pltpu.SemaphoreType.DMA for _ in range(NBUF)],
            [pltpu.SemaphoreType.DMA for _ in range(NBUF)],
        ],
        compiler_params=pltpu.CompilerParams(
            use_tc_tiling_on_sc=False, needs_layout_passes=False
        ),
    )
    def gather_kernel(table_hbm, idx_hbm, out_hbm, idx_v, gbufs, tbufs, gsems, osems):
        wid = lax.axis_index("s") * 2 + lax.axis_index("c")
        g0 = wid * g_per_w
        pltpu.sync_copy(idx_hbm.at[pl.ds(g0 * GROUP, n_per_w)], idx_v)

        nb = B // GROUP

        def g_start(g, b):
            for k in range(GRP_CHUNKS):
                pltpu.async_copy(
                    table_hbm.at[idx_v.at[pl.ds(g * GROUP + k * CHUNK, CHUNK)]],
                    gbufs[b].at[pl.ds(k * CHUNK, CHUNK)],
                    gsems[b],
                )

        def g_wait(b):
            for k in range(GRP_CHUNKS):
                pltpu.make_async_copy(
                    table_hbm.at[idx_v.at[pl.ds(0, CHUNK)]],
                    gbufs[b].at[pl.ds(k * CHUNK, CHUNK)],
                    gsems[b],
                ).wait()

        def o_start(gq, b):
            s = gq // nb
            b0 = (gq % nb) * GROUP
            pltpu.async_copy(
                tbufs[b], out_hbm.at[s, :, pl.ds(b0, GROUP)], osems[b]
            )

        def o_wait(b):
            pltpu.make_async_copy(
                tbufs[b], out_hbm.at[0, :, pl.ds(0, GROUP)], osems[b]
            ).wait()

        def transpose(b):
            gbuf, tbuf = gbufs[b], tbufs[b]

            def tbody(grp, carry):
                r = lax.iota(jnp.int32, LANES) + grp * LANES
                for c in range(EMB):
                    vals = plsc.load_gather(
                        gbuf, [r, jnp.full((LANES,), c, jnp.int32)]
                    )
                    tbuf[c, pl.ds(grp * LANES, LANES)] = vals
                return carry

            lax.fori_loop(0, GROUP // LANES, tbody, 0)

        for b in range(NBUF):
            g_start(b, b)

        def body(i, carry):
            gg = i * NBUF
            for b in range(NBUF):
                g = gg + b
                g_wait(b)

                @pl.when(i > 0)
                def _():
                    o_wait(b)

                transpose(b)

                @pl.when(g + NBUF < g_per_w)
                def _():
                    g_start(g + NBUF, b)

                o_start(g0 + g, b)
            return carry

        lax.fori_loop(0, g_per_w // NBUF, body, 0)
        for b in range(NBUF):
            o_wait(b)

    return gather_kernel


def kernel(emb_table, indices):
    Bn, Sn = indices.shape
    idx_flat = indices.T.reshape(-1).astype(jnp.int32)
    out_t = _build(Sn, Bn)(emb_table, idx_flat)
    return out_t.transpose(2, 0, 1)


# speedup vs baseline: 6.0097x; 1.7257x over previous
import functools

import jax
import jax.numpy as jnp
from jax import lax
from jax.experimental import pallas as pl
from jax.experimental.pallas import tpu as pltpu
from jax.experimental.pallas import tpu_sc as plsc

EMB = 32
CHUNK = 128
GRP_CHUNKS = 4
GROUP = CHUNK * GRP_CHUNKS
NBUF = 2
NUM_WORKERS = 32
LANES = 16


@functools.cache
def _build(S, B):
    ng = (S * B) // GROUP
    assert ng % (NUM_WORKERS * NBUF) == 0 and B % GROUP == 0
    g_per_w = ng // NUM_WORKERS
    n_per_w = g_per_w * GROUP
    mesh = plsc.VectorSubcoreMesh(core_axis_name="c", subcore_axis_name="s")

    @functools.partial(
        pl.kernel,
        mesh=mesh,
        out_type=jax.ShapeDtypeStruct((S, EMB, B), jnp.float32),
        scratch_types=[
            pltpu.VMEM((n_per_w,), jnp.int32),
            [pltpu.VMEM((GROUP, EMB), jnp.float32) for _ in range(NBUF)],
            [pltpu.VMEM((EMB, GROUP), jnp.float32) for _ in range(NBUF)],
            [pltpu.SemaphoreType.DMA for _ in range(NBUF)],
            [pltpu.SemaphoreType.DMA for _ in range(NBUF)],
        ],
        compiler_params=pltpu.CompilerParams(
            use_tc_tiling_on_sc=False, needs_layout_passes=False
        ),
    )
    def gather_kernel(table_hbm, idx_hbm, out_hbm, idx_v, gbufs, tbufs, gsems, osems):
        wid = lax.axis_index("s") * 2 + lax.axis_index("c")
        g0 = wid * g_per_w
        pltpu.sync_copy(idx_hbm.at[pl.ds(g0 * GROUP, n_per_w)], idx_v)

        nb = B // GROUP

        def g_start(g, b):
            for k in range(GRP_CHUNKS):
                pltpu.async_copy(
                    table_hbm.at[idx_v.at[pl.ds(g * GROUP + k * CHUNK, CHUNK)]],
                    gbufs[b].at[pl.ds(k * CHUNK, CHUNK)],
                    gsems[b],
                )

        def g_wait(b):
            for k in range(GRP_CHUNKS):
                pltpu.make_async_copy(
                    table_hbm.at[idx_v.at[pl.ds(0, CHUNK)]],
                    gbufs[b].at[pl.ds(k * CHUNK, CHUNK)],
                    gsems[b],
                ).wait()

        def o_start(gq, b):
            s = gq // nb
            b0 = (gq % nb) * GROUP
            pltpu.async_copy(
                tbufs[b], out_hbm.at[s, :, pl.ds(b0, GROUP)], osems[b]
            )

        def o_wait(b):
            pltpu.make_async_copy(
                tbufs[b], out_hbm.at[0, :, pl.ds(0, GROUP)], osems[b]
            ).wait()

        def transpose(b):
            gbuf, tbuf = gbufs[b], tbufs[b]

            def tbody(grp, carry):
                lanes = lax.iota(jnp.int32, LANES)
                r = lanes + grp * LANES
                for c in range(EMB):
                    diag = (lanes + c) & (EMB - 1)
                    vals = plsc.load_gather(gbuf, [r, diag])
                    plsc.store_scatter(tbuf, [diag, r], vals)
                return carry

            lax.fori_loop(0, GROUP // LANES, tbody, 0)

        for b in range(NBUF):
            g_start(b, b)

        def body(i, carry):
            gg = i * NBUF
            for b in range(NBUF):
                g = gg + b
                g_wait(b)

                @pl.when(i > 0)
                def _():
                    o_wait(b)

                transpose(b)

                @pl.when(g + NBUF < g_per_w)
                def _():
                    g_start(g + NBUF, b)

                o_start(g0 + g, b)
            return carry

        lax.fori_loop(0, g_per_w // NBUF, body, 0)
        for b in range(NBUF):
            o_wait(b)

    return gather_kernel


def kernel(emb_table, indices):
    Bn, Sn = indices.shape
    idx_flat = indices.T.reshape(-1).astype(jnp.int32)
    out_t = _build(Sn, Bn)(emb_table, idx_flat)
    return out_t.transpose(2, 0, 1)


# speedup vs baseline: 6.8706x; 1.1432x over previous
import functools

import jax
import jax.numpy as jnp
from jax import lax
from jax.experimental import pallas as pl
from jax.experimental.pallas import tpu as pltpu
from jax.experimental.pallas import tpu_sc as plsc

EMB = 32
CHUNK = 128
GRP_CHUNKS = 4
GROUP = CHUNK * GRP_CHUNKS
NBUF = 2
NUM_WORKERS = 32
LANES = 16


@functools.cache
def _build(S, B):
    ng = (S * B) // GROUP
    assert ng % (NUM_WORKERS * NBUF) == 0 and B % GROUP == 0
    g_per_w = ng // NUM_WORKERS
    n_per_w = g_per_w * GROUP
    mesh = plsc.VectorSubcoreMesh(core_axis_name="c", subcore_axis_name="s")

    @functools.partial(
        pl.kernel,
        mesh=mesh,
        out_type=jax.ShapeDtypeStruct((S, EMB // 8, (B // 128) * 8 * 128), jnp.float32),
        scratch_types=[
            pltpu.VMEM((n_per_w,), jnp.int32),
            [pltpu.VMEM((GROUP, EMB), jnp.float32) for _ in range(NBUF)],
            [pltpu.VMEM((EMB // 8, GROUP * 8), jnp.float32) for _ in range(NBUF)],
            [pltpu.SemaphoreType.DMA for _ in range(NBUF)],
            [pltpu.SemaphoreType.DMA for _ in range(NBUF)],
        ],
        compiler_params=pltpu.CompilerParams(
            use_tc_tiling_on_sc=False, needs_layout_passes=False
        ),
    )
    def gather_kernel(table_hbm, idx_hbm, out_hbm, idx_v, gbufs, tbufs, gsems, osems):
        wid = lax.axis_index("s") * 2 + lax.axis_index("c")
        g0 = wid * g_per_w
        pltpu.sync_copy(idx_hbm.at[pl.ds(g0 * GROUP, n_per_w)], idx_v)

        nb = B // GROUP

        def g_start(g, b):
            for k in range(GRP_CHUNKS):
                pltpu.async_copy(
                    table_hbm.at[idx_v.at[pl.ds(g * GROUP + k * CHUNK, CHUNK)]],
                    gbufs[b].at[pl.ds(k * CHUNK, CHUNK)],
                    gsems[b],
                )

        def g_wait(b):
            for k in range(GRP_CHUNKS):
                pltpu.make_async_copy(
                    table_hbm.at[idx_v.at[pl.ds(0, CHUNK)]],
                    gbufs[b].at[pl.ds(k * CHUNK, CHUNK)],
                    gsems[b],
                ).wait()

        def o_start(gq, b):
            s = gq // nb
            b0 = (gq % nb) * (GROUP * 8)
            pltpu.async_copy(
                tbufs[b], out_hbm.at[s, :, pl.ds(b0, GROUP * 8)], osems[b]
            )

        def o_wait(b):
            pltpu.make_async_copy(
                tbufs[b], out_hbm.at[0, :, pl.ds(0, GROUP * 8)], osems[b]
            ).wait()

        def transpose(b):
            gbuf, tbuf = gbufs[b], tbufs[b]

            def tbody(grp, carry):
                lanes = lax.iota(jnp.int32, LANES)
                r = lanes + grp * LANES
                rmap = ((r >> 7) << 10) + (r & 127)
                for c in range(EMB):
                    diag = (lanes + c) & (EMB - 1)
                    vals = plsc.load_gather(gbuf, [r, diag])
                    plsc.store_scatter(
                        tbuf, [diag >> 3, rmap + ((diag & 7) << 7)], vals
                    )
                return carry

            lax.fori_loop(0, GROUP // LANES, tbody, 0)

        for b in range(NBUF):
            g_start(b, b)

        def body(i, carry):
            gg = i * NBUF
            for b in range(NBUF):
                g = gg + b
                g_wait(b)

                @pl.when(i > 0)
                def _():
                    o_wait(b)

                transpose(b)

                @pl.when(g + NBUF < g_per_w)
                def _():
                    g_start(g + NBUF, b)

                o_start(g0 + g, b)
            return carry

        lax.fori_loop(0, g_per_w // NBUF, body, 0)
        for b in range(NBUF):
            o_wait(b)

    return gather_kernel


def kernel(emb_table, indices):
    Bn, Sn = indices.shape
    idx_flat = indices.T.reshape(-1).astype(jnp.int32)
    out_t = _build(Sn, Bn)(emb_table, idx_flat)
    out5 = out_t.reshape(Sn, EMB // 8, Bn // 128, 8, 128)
    return out5.transpose(2, 4, 0, 1, 3).reshape(Bn, Sn, EMB)


# speedup vs baseline: 7.3069x; 1.0635x over previous
import functools

import jax
import jax.numpy as jnp
from jax import lax
from jax.experimental import pallas as pl
from jax.experimental.pallas import tpu as pltpu
from jax.experimental.pallas import tpu_sc as plsc

EMB = 32
CHUNK = 128
GRP_CHUNKS = 1
GROUP = CHUNK * GRP_CHUNKS
NBUF = 8
NUM_WORKERS = 32
LANES = 16


@functools.cache
def _build(S, B):
    ng = (S * B) // GROUP
    assert ng % (NUM_WORKERS * NBUF) == 0 and B % GROUP == 0
    g_per_w = ng // NUM_WORKERS
    n_per_w = g_per_w * GROUP
    mesh = plsc.VectorSubcoreMesh(core_axis_name="c", subcore_axis_name="s")

    @functools.partial(
        pl.kernel,
        mesh=mesh,
        out_type=jax.ShapeDtypeStruct((S, EMB // 8, (B // 128) * 8 * 128), jnp.float32),
        scratch_types=[
            pltpu.VMEM((n_per_w,), jnp.int32),
            [pltpu.VMEM((GROUP, EMB), jnp.float32) for _ in range(NBUF)],
            [pltpu.VMEM((EMB // 8, GROUP * 8), jnp.float32) for _ in range(NBUF)],
            [pltpu.SemaphoreType.DMA for _ in range(NBUF)],
            [pltpu.SemaphoreType.DMA for _ in range(NBUF)],
        ],
        compiler_params=pltpu.CompilerParams(
            use_tc_tiling_on_sc=False, needs_layout_passes=False
        ),
    )
    def gather_kernel(table_hbm, idx_hbm, out_hbm, idx_v, gbufs, tbufs, gsems, osems):
        wid = lax.axis_index("s") * 2 + lax.axis_index("c")
        g0 = wid * g_per_w
        pltpu.sync_copy(idx_hbm.at[pl.ds(g0 * GROUP, n_per_w)], idx_v)

        nb = B // GROUP

        def g_start(g, b):
            for k in range(GRP_CHUNKS):
                pltpu.async_copy(
                    table_hbm.at[idx_v.at[pl.ds(g * GROUP + k * CHUNK, CHUNK)]],
                    gbufs[b].at[pl.ds(k * CHUNK, CHUNK)],
                    gsems[b],
                )

        def g_wait(b):
            for k in range(GRP_CHUNKS):
                pltpu.make_async_copy(
                    table_hbm.at[idx_v.at[pl.ds(0, CHUNK)]],
                    gbufs[b].at[pl.ds(k * CHUNK, CHUNK)],
                    gsems[b],
                ).wait()

        def o_start(gq, b):
            s = gq // nb
            b0 = (gq % nb) * (GROUP * 8)
            pltpu.async_copy(
                tbufs[b], out_hbm.at[s, :, pl.ds(b0, GROUP * 8)], osems[b]
            )

        def o_wait(b):
            pltpu.make_async_copy(
                tbufs[b], out_hbm.at[0, :, pl.ds(0, GROUP * 8)], osems[b]
            ).wait()

        def transpose(b):
            gbuf, tbuf = gbufs[b], tbufs[b]

            def tbody(grp, carry):
                lanes = lax.iota(jnp.int32, LANES)
                r = lanes + grp * LANES
                rmap = ((r >> 7) << 10) + (r & 127)
                for c in range(EMB):
                    diag = (lanes + c) & (EMB - 1)
                    vals = plsc.load_gather(gbuf, [r, diag])
                    plsc.store_scatter(
                        tbuf, [diag >> 3, rmap + ((diag & 7) << 7)], vals
                    )
                return carry

            lax.fori_loop(0, GROUP // LANES, tbody, 0)

        for b in range(NBUF):
            g_start(b, b)

        def body(i, carry):
            gg = i * NBUF
            for b in range(NBUF):
                g = gg + b
                g_wait(b)

                @pl.when(i > 0)
                def _():
                    o_wait(b)

                transpose(b)

                @pl.when(g + NBUF < g_per_w)
                def _():
                    g_start(g + NBUF, b)

                o_start(g0 + g, b)
            return carry

        lax.fori_loop(0, g_per_w // NBUF, body, 0)
        for b in range(NBUF):
            o_wait(b)

    return gather_kernel


def kernel(emb_table, indices):
    Bn, Sn = indices.shape
    idx_flat = indices.T.reshape(-1).astype(jnp.int32)
    out_t = _build(Sn, Bn)(emb_table, idx_flat)
    out5 = out_t.reshape(Sn, EMB // 8, Bn // 128, 8, 128)
    return out5.transpose(2, 4, 0, 1, 3).reshape(Bn, Sn, EMB)
